# Initial kernel scaffold; baseline (speedup 1.0000x reference)
import jax
import jax.numpy as jnp
from jax.experimental import pallas as pl


def kernel(pack_data, batch_sizes, context, context_mask, Wq, Wk, Wz, Wr, Wn, Uz, Ur, Un, bz, br, bn):
    raise NotImplementedError("write your pallas kernel here")


# speedup vs baseline: 4.7122x; 4.7122x over previous
import numpy as np
import jax
import jax.numpy as jnp
from jax.experimental import pallas as pl
from jax.experimental.pallas import tpu as pltpu

D = 512
H = 512
CD = 512
L = 128
B = 16
T = 512
TOTAL = 4352
PAD = TOTAL + B
SCALE = 1.0 / np.sqrt(H)


def _matmul_kernel(a_ref, b_ref, o_ref):
    o_ref[...] = jnp.dot(a_ref[...], b_ref[...],
                         preferred_element_type=jnp.float32)


def _proj_kernel(a_ref, b_ref, bias_ref, o_ref):
    o_ref[...] = jnp.dot(a_ref[...], b_ref[...],
                         preferred_element_type=jnp.float32) + bias_ref[...]


def _loop_kernel(x_ref, k_ref, ctx_ref, madd_ref, wq_ref, ucat_ref, un_ref,
                 wa_ref, out_ref, hf_ref, h_scr):
    h_scr[...] = jnp.zeros((B, H), jnp.float32)
    lane = jax.lax.broadcasted_iota(jnp.int32, (B, 1), 0)
    row24 = jax.lax.broadcasted_iota(jnp.int32, (24, 1), 0)

    def step(t, off):
        b = B - t // 32
        a8 = off // 8 * 8
        d = off - a8
        h = h_scr[...]
        q = jnp.dot(h, wq_ref[...], preferred_element_type=jnp.float32)
        s = jnp.sum(q[:, None, :] * k_ref[...], axis=-1) * SCALE
        s = s + madd_ref[...]
        m = jnp.max(s, axis=-1, keepdims=True)
        e = jnp.exp(s - m)
        w = e / jnp.sum(e, axis=-1, keepdims=True)
        attn = jnp.sum(w[:, :, None] * ctx_ref[...], axis=1)
        xwin = pltpu.roll(x_ref[pl.ds(a8, 24), :], (24 - d) % 24, axis=0)
        g = xwin[:B] + jnp.dot(
            attn, wa_ref[...], preferred_element_type=jnp.float32)
        zr = jax.nn.sigmoid(
            g[:, : 2 * H]
            + jnp.dot(h, ucat_ref[...], preferred_element_type=jnp.float32))
        z = zr[:, :H]
        r = zr[:, H:]
        n = jnp.tanh(g[:, 2 * H:] + jnp.dot(
            r * h, un_ref[...], preferred_element_type=jnp.float32))
        hn = (1.0 - z) * n + z * h
        hsel = jnp.where(lane < b, hn, h)
        h_scr[...] = hsel
        owin = pltpu.roll(
            jnp.concatenate([hsel, jnp.zeros((8, H), jnp.float32)], axis=0),
            d, axis=0)
        keep = (row24 >= d) & (row24 < d + B)
        out_ref[pl.ds(a8, 24), :] = jnp.where(
            keep, owin, out_ref[pl.ds(a8, 24), :])
        return off + b

    jax.lax.fori_loop(0, T, step, jnp.int32(0))
    hf_ref[...] = h_scr[...][None]


def kernel(pack_data, batch_sizes, context, context_mask, Wq, Wk, Wz, Wr, Wn,
           Uz, Ur, Un, bz, br, bn):
    f32 = jnp.float32
    pack_pad = jnp.zeros((PAD, D), f32).at[:TOTAL].set(pack_data)
    wcat = jnp.concatenate([Wz[:D], Wr[:D], Wn[:D]], axis=1)
    bcat = jnp.concatenate([bz, br, bn])[None, :]
    wa = jnp.concatenate([Wz[D:], Wr[D:], Wn[D:]], axis=1)
    ucat = jnp.concatenate([Uz, Ur], axis=1)
    madd = jnp.where(context_mask, 0.0, -1e9).astype(f32)

    K = pl.pallas_call(
        _matmul_kernel,
        out_shape=jax.ShapeDtypeStruct((B * L, H), f32),
    )(context.reshape(B * L, CD), Wk)

    X = pl.pallas_call(
        _proj_kernel,
        out_shape=jax.ShapeDtypeStruct((PAD, 3 * H), f32),
        compiler_params=pltpu.CompilerParams(vmem_limit_bytes=100 * 2**20),
    )(pack_pad, wcat, bcat)

    out_pad, hidden_final = pl.pallas_call(
        _loop_kernel,
        out_shape=(jax.ShapeDtypeStruct((PAD, H), f32),
                   jax.ShapeDtypeStruct((1, B, H), f32)),
        scratch_shapes=[pltpu.VMEM((B, H), f32)],
        compiler_params=pltpu.CompilerParams(vmem_limit_bytes=110 * 2**20),
    )(X, K.reshape(B, L, H), context, madd, Wq, ucat, Un, wa)

    return out_pad[:TOTAL], hidden_final


# speedup vs baseline: 5.2266x; 1.1092x over previous
import numpy as np
import jax
import jax.numpy as jnp
from jax.experimental import pallas as pl
from jax.experimental.pallas import tpu as pltpu

D = 512
H = 512
CD = 512
L = 128
B = 16
T = 512
TOTAL = 4352
PAD = TOTAL + B
SCALE = 1.0 / np.sqrt(H)


def _matmul_kernel(a_ref, b_ref, o_ref):
    o_ref[...] = jnp.dot(a_ref[...], b_ref[...],
                         preferred_element_type=jnp.float32
                         ).astype(jnp.bfloat16)


def _proj_kernel(a_ref, b_ref, bias_ref, o_ref):
    o_ref[...] = jnp.dot(a_ref[...], b_ref[...],
                         preferred_element_type=jnp.float32) + bias_ref[...]


def _loop_kernel(x_ref, k_ref, ctx_ref, madd_ref, wq_ref, ucat_ref, un_ref,
                 wa_ref, out_ref, hf_ref, h_scr):
    h_scr[...] = jnp.zeros((B, H), jnp.float32)

    def make_step(nb):
        win = nb + 8
        lane = jax.lax.broadcasted_iota(jnp.int32, (nb, 1), 0)
        roww = jax.lax.broadcasted_iota(jnp.int32, (win, 1), 0)

        def step(t, off):
            b = B - t // 32
            a8 = off // 8 * 8
            d = off - a8
            h = h_scr[0:nb, :]
            q = jnp.dot(h, wq_ref[...], preferred_element_type=jnp.float32)
            s = jnp.sum(q.astype(jnp.bfloat16)[:, None, :] * k_ref[0:nb],
                        axis=-1, dtype=jnp.float32) * SCALE
            s = s + madd_ref[0:nb]
            m = jnp.max(s, axis=-1, keepdims=True)
            e = jnp.exp(s - m)
            w = (e / jnp.sum(e, axis=-1, keepdims=True)).astype(jnp.bfloat16)
            attn = jnp.sum(w[:, :, None] * ctx_ref[0:nb], axis=1,
                           dtype=jnp.float32)
            xwin = pltpu.roll(x_ref[pl.ds(a8, win), :], (win - d) % win,
                              axis=0)
            g = xwin[:nb] + jnp.dot(
                attn, wa_ref[...], preferred_element_type=jnp.float32)
            zr = jax.nn.sigmoid(
                g[:, : 2 * H]
                + jnp.dot(h, ucat_ref[...],
                          preferred_element_type=jnp.float32))
            z = zr[:, :H]
            r = zr[:, H:]
            n = jnp.tanh(g[:, 2 * H:] + jnp.dot(
                r * h, un_ref[...], preferred_element_type=jnp.float32))
            hn = (1.0 - z) * n + z * h
            hsel = jnp.where(lane < b, hn, h)
            h_scr[0:nb, :] = hsel
            owin = pltpu.roll(
                jnp.concatenate([hsel, jnp.zeros((8, H), jnp.float32)],
                                axis=0), d, axis=0)
            keep = (roww >= d) & (roww < d + nb)
            out_ref[pl.ds(a8, win), :] = jnp.where(
                keep, owin, out_ref[pl.ds(a8, win), :])
            return off + b

        return step

    off = jax.lax.fori_loop(0, T // 2, make_step(B), jnp.int32(0))
    jax.lax.fori_loop(T // 2, T, make_step(B // 2), off)
    hf_ref[...] = h_scr[...][None]


def kernel(pack_data, batch_sizes, context, context_mask, Wq, Wk, Wz, Wr, Wn,
           Uz, Ur, Un, bz, br, bn):
    f32 = jnp.float32
    pack_pad = jnp.zeros((PAD, D), f32).at[:TOTAL].set(pack_data)
    wcat = jnp.concatenate([Wz[:D], Wr[:D], Wn[:D]], axis=1)
    bcat = jnp.concatenate([bz, br, bn])[None, :]
    wa = jnp.concatenate([Wz[D:], Wr[D:], Wn[D:]], axis=1)
    ucat = jnp.concatenate([Uz, Ur], axis=1)
    madd = jnp.where(context_mask, 0.0, -1e9).astype(f32)

    K = pl.pallas_call(
        _matmul_kernel,
        out_shape=jax.ShapeDtypeStruct((B * L, H), jnp.bfloat16),
    )(context.reshape(B * L, CD), Wk)

    X = pl.pallas_call(
        _proj_kernel,
        out_shape=jax.ShapeDtypeStruct((PAD, 3 * H), f32),
        compiler_params=pltpu.CompilerParams(vmem_limit_bytes=100 * 2**20),
    )(pack_pad, wcat, bcat)

    out_pad, hidden_final = pl.pallas_call(
        _loop_kernel,
        out_shape=(jax.ShapeDtypeStruct((PAD, H), f32),
                   jax.ShapeDtypeStruct((1, B, H), f32)),
        scratch_shapes=[pltpu.VMEM((B, H), f32)],
        compiler_params=pltpu.CompilerParams(vmem_limit_bytes=110 * 2**20),
    )(X, K.reshape(B, L, H), context.astype(jnp.bfloat16), madd, Wq, ucat,
      Un, wa)

    return out_pad[:TOTAL], hidden_final


# speedup vs baseline: 7.7606x; 1.4848x over previous
import numpy as np
import jax
import jax.numpy as jnp
from jax.experimental import pallas as pl
from jax.experimental.pallas import tpu as pltpu

D = 512
H = 512
CD = 512
L = 128
B = 16
T = 512
TOTAL = 4352
PAD = TOTAL + B
SCALE = 1.0 / np.sqrt(H)


def _matmul_kernel(a_ref, b_ref, o_ref):
    o_ref[...] = jax.lax.dot_general(
        a_ref[...], b_ref[...], (((0,), (1,)), ((), ())),
        preferred_element_type=jnp.float32).astype(jnp.bfloat16)


def _proj_kernel(a_ref, b_ref, bias_ref, o_ref):
    o_ref[...] = jnp.dot(a_ref[...], b_ref[...],
                         preferred_element_type=jnp.float32) + bias_ref[...]


def _loop_kernel(x_ref, kt_ref, ctx2_ref, madd_ref, wq_ref, ucat_ref, un_ref,
                 wa_ref, out_ref, hf_ref, h_scr):
    h_scr[...] = jnp.zeros((B, H), jnp.float32)

    def make_step(nb):
        win = nb + 8
        lane = jax.lax.broadcasted_iota(jnp.int32, (nb, 1), 0)
        roww = jax.lax.broadcasted_iota(jnp.int32, (win, 1), 0)

        def step(t, off):
            b = B - t // 32
            a8 = off // 8 * 8
            d = off - a8
            h = h_scr[0:nb, :]
            q = jnp.dot(h, wq_ref[...], preferred_element_type=jnp.float32)
            s = jnp.dot(q.astype(jnp.bfloat16), kt_ref[:, 0:nb * L],
                        preferred_element_type=jnp.float32) * SCALE
            s = s + madd_ref[0:nb, 0:nb * L]
            m = jnp.max(s, axis=-1, keepdims=True)
            e = jnp.exp(s - m)
            w = (e / jnp.sum(e, axis=-1, keepdims=True)).astype(jnp.bfloat16)
            attn = jnp.dot(w, ctx2_ref[0:nb * L, :],
                           preferred_element_type=jnp.float32)
            xwin = pltpu.roll(x_ref[pl.ds(a8, win), :], (win - d) % win,
                              axis=0)
            g = xwin[:nb] + jnp.dot(
                attn, wa_ref[...], preferred_element_type=jnp.float32)
            zr = jax.nn.sigmoid(
                g[:, : 2 * H]
                + jnp.dot(h, ucat_ref[...],
                          preferred_element_type=jnp.float32))
            z = zr[:, :H]
            r = zr[:, H:]
            n = jnp.tanh(g[:, 2 * H:] + jnp.dot(
                r * h, un_ref[...], preferred_element_type=jnp.float32))
            hn = (1.0 - z) * n + z * h
            hsel = jnp.where(lane < b, hn, h)
            h_scr[0:nb, :] = hsel
            owin = pltpu.roll(
                jnp.concatenate([hsel, jnp.zeros((8, H), jnp.float32)],
                                axis=0), d, axis=0)
            keep = (roww >= d) & (roww < d + nb)
            out_ref[pl.ds(a8, win), :] = jnp.where(
                keep, owin, out_ref[pl.ds(a8, win), :])
            return off + b

        return step

    off = jax.lax.fori_loop(0, T // 2, make_step(B), jnp.int32(0))
    jax.lax.fori_loop(T // 2, T, make_step(B // 2), off)
    hf_ref[...] = h_scr[...][None]


def kernel(pack_data, batch_sizes, context, context_mask, Wq, Wk, Wz, Wr, Wn,
           Uz, Ur, Un, bz, br, bn):
    f32 = jnp.float32
    pack_pad = jnp.zeros((PAD, D), f32).at[:TOTAL].set(pack_data)
    wcat = jnp.concatenate([Wz[:D], Wr[:D], Wn[:D]], axis=1)
    bcat = jnp.concatenate([bz, br, bn])[None, :]
    wa = jnp.concatenate([Wz[D:], Wr[D:], Wn[D:]], axis=1)
    ucat = jnp.concatenate([Uz, Ur], axis=1)
    madd1 = jnp.where(context_mask, 0.0, -1e9).astype(f32)
    madd = jnp.where(jnp.eye(B, dtype=bool)[:, :, None],
                     madd1[:, None, :], -1e9).reshape(B, B * L).astype(f32)

    KT = pl.pallas_call(
        _matmul_kernel,
        out_shape=jax.ShapeDtypeStruct((H, B * L), jnp.bfloat16),
    )(Wk, context.reshape(B * L, CD))

    X = pl.pallas_call(
        _proj_kernel,
        out_shape=jax.ShapeDtypeStruct((PAD, 3 * H), f32),
        compiler_params=pltpu.CompilerParams(vmem_limit_bytes=100 * 2**20),
    )(pack_pad, wcat, bcat)

    out_pad, hidden_final = pl.pallas_call(
        _loop_kernel,
        out_shape=(jax.ShapeDtypeStruct((PAD, H), f32),
                   jax.ShapeDtypeStruct((1, B, H), f32)),
        scratch_shapes=[pltpu.VMEM((B, H), f32)],
        compiler_params=pltpu.CompilerParams(vmem_limit_bytes=110 * 2**20),
    )(X, KT, context.reshape(B * L, CD).astype(jnp.bfloat16), madd, Wq, ucat,
      Un, wa)

    return out_pad[:TOTAL], hidden_final


# speedup vs baseline: 8.6529x; 1.1150x over previous
import numpy as np
import jax
import jax.numpy as jnp
from jax.experimental import pallas as pl
from jax.experimental.pallas import tpu as pltpu

D = 512
H = 512
CD = 512
L = 128
B = 16
T = 512
TOTAL = 4352
PAD = TOTAL + B
SCALE = 1.0 / np.sqrt(H)


def _qkt_kernel(wq_ref, wk_ref, c2_ref, o_ref):
    kt = jax.lax.dot_general(
        wk_ref[...], c2_ref[...], (((0,), (1,)), ((), ())),
        preferred_element_type=jnp.float32)
    o_ref[...] = (SCALE * jnp.dot(
        wq_ref[...], kt, preferred_element_type=jnp.float32)
                  ).astype(jnp.bfloat16)


def _proj_kernel(a_ref, b_ref, bias_ref, o_ref):
    o_ref[...] = jnp.dot(a_ref[...], b_ref[...],
                         preferred_element_type=jnp.float32) + bias_ref[...]


def _loop_kernel(x_ref, qkt_ref, ctx2_ref, madd_ref, ucat_ref, un_ref,
                 wa_ref, out_ref, hf_ref):

    def make_pair(nb):
        win = nb + 8
        lane = jax.lax.broadcasted_iota(jnp.int32, (nb, 1), 0)
        roww = jax.lax.broadcasted_iota(jnp.int32, (win, 1), 0)

        def substep(t, off, h):
            b = B - t // 32
            a8 = off // 8 * 8
            d = off - a8
            s = jnp.dot(h.astype(jnp.bfloat16), qkt_ref[:, 0:nb * L],
                        preferred_element_type=jnp.float32)
            s = s + madd_ref[0:nb, 0:nb * L]
            m = jnp.max(s, axis=-1, keepdims=True)
            e = jnp.exp(s - m)
            w = (e / jnp.sum(e, axis=-1, keepdims=True)).astype(jnp.bfloat16)
            attn = jnp.dot(w, ctx2_ref[0:nb * L, :],
                           preferred_element_type=jnp.float32)
            xwin = pltpu.roll(x_ref[pl.ds(a8, win), :], (win - d) % win,
                              axis=0)
            g = xwin[:nb] + jnp.dot(
                attn, wa_ref[...], preferred_element_type=jnp.float32)
            zr = jax.nn.sigmoid(
                g[:, : 2 * H]
                + jnp.dot(h, ucat_ref[...],
                          preferred_element_type=jnp.float32))
            z = zr[:, :H]
            r = zr[:, H:]
            n = jnp.tanh(g[:, 2 * H:] + jnp.dot(
                r * h, un_ref[...], preferred_element_type=jnp.float32))
            hn = (1.0 - z) * n + z * h
            hsel = jnp.where(lane < b, hn, h)
            owin = pltpu.roll(
                jnp.concatenate([hsel, jnp.zeros((8, H), jnp.float32)],
                                axis=0), d, axis=0)
            keep = (roww >= d) & (roww < d + nb)
            out_ref[pl.ds(a8, win), :] = jnp.where(
                keep, owin, out_ref[pl.ds(a8, win), :])
            return off + b, hsel

        def pair(it, carry):
            off, h = carry
            off, h = substep(2 * it, off, h)
            off, h = substep(2 * it + 1, off, h)
            return off, h

        return pair

    h0 = jnp.zeros((B, H), jnp.float32)
    off, h = jax.lax.fori_loop(0, T // 4, make_pair(B), (jnp.int32(0), h0))
    hf_ref[0, B // 2:, :] = h[B // 2:]
    _, h8 = jax.lax.fori_loop(T // 4, T // 2, make_pair(B // 2),
                              (off, h[: B // 2]))
    hf_ref[0, 0: B // 2, :] = h8


def kernel(pack_data, batch_sizes, context, context_mask, Wq, Wk, Wz, Wr, Wn,
           Uz, Ur, Un, bz, br, bn):
    f32 = jnp.float32
    pack_pad = jnp.zeros((PAD, D), f32).at[:TOTAL].set(pack_data)
    wcat = jnp.concatenate([Wz[:D], Wr[:D], Wn[:D]], axis=1)
    bcat = jnp.concatenate([bz, br, bn])[None, :]
    wa = jnp.concatenate([Wz[D:], Wr[D:], Wn[D:]], axis=1)
    ucat = jnp.concatenate([Uz, Ur], axis=1)
    ctx2 = context.reshape(B * L, CD)
    madd1 = jnp.where(context_mask, 0.0, -1e9).astype(f32)
    madd = jnp.where(jnp.eye(B, dtype=bool)[:, :, None],
                     madd1[:, None, :], -1e9).reshape(B, B * L).astype(f32)

    QKT = pl.pallas_call(
        _qkt_kernel,
        out_shape=jax.ShapeDtypeStruct((H, B * L), jnp.bfloat16),
    )(Wq, Wk, ctx2)

    X = pl.pallas_call(
        _proj_kernel,
        out_shape=jax.ShapeDtypeStruct((PAD, 3 * H), f32),
        compiler_params=pltpu.CompilerParams(vmem_limit_bytes=100 * 2**20),
    )(pack_pad, wcat, bcat)

    out_pad, hidden_final = pl.pallas_call(
        _loop_kernel,
        out_shape=(jax.ShapeDtypeStruct((PAD, H), f32),
                   jax.ShapeDtypeStruct((1, B, H), f32)),
        compiler_params=pltpu.CompilerParams(vmem_limit_bytes=110 * 2**20),
    )(X, QKT, ctx2.astype(jnp.bfloat16), madd, ucat, Un, wa)

    return out_pad[:TOTAL], hidden_final


# speedup vs baseline: 9.9225x; 1.1467x over previous
import numpy as np
import jax
import jax.numpy as jnp
from jax.experimental import pallas as pl
from jax.experimental.pallas import tpu as pltpu

D = 512
H = 512
CD = 512
L = 128
B = 16
T = 512
TOTAL = 4352
PAD = TOTAL + B
SCALE = 1.0 / np.sqrt(H)


def _qkt_kernel(wq_ref, wk_ref, c2_ref, o_ref):
    kt = jax.lax.dot_general(
        wk_ref[...], c2_ref[...], (((0,), (1,)), ((), ())),
        preferred_element_type=jnp.float32)
    o_ref[...] = (SCALE * jnp.dot(
        wq_ref[...], kt, preferred_element_type=jnp.float32)
                  ).astype(jnp.bfloat16)


def _proj_kernel(a_ref, b_ref, bias_ref, o_ref):
    o_ref[...] = jnp.dot(a_ref[...], b_ref[...],
                         preferred_element_type=jnp.float32) + bias_ref[...]


def _loop_kernel(x_ref, hm16_ref, hm8_ref, ctx2_ref, madd_ref, un_ref,
                 wa_ref, out_ref, hf_ref):

    def make_pair(nb):
        win = nb + 8
        hm_ref = hm16_ref if nb == B else hm8_ref
        lane = jax.lax.broadcasted_iota(jnp.int32, (nb, 1), 0)
        roww = jax.lax.broadcasted_iota(jnp.int32, (win, 1), 0)

        def substep(t, off, h):
            b = B - t // 32
            a8 = off // 8 * 8
            d = off - a8
            hm = jnp.dot(h.astype(jnp.bfloat16), hm_ref[...],
                         preferred_element_type=jnp.float32)
            s = hm[:, 0:nb * L] + madd_ref[0:nb, 0:nb * L]
            m = jnp.max(s, axis=-1, keepdims=True)
            e = jnp.exp(s - m)
            w = (e / jnp.sum(e, axis=-1, keepdims=True)).astype(jnp.bfloat16)
            attn = jnp.dot(w, ctx2_ref[0:nb * L, :],
                           preferred_element_type=jnp.float32)
            xwin = pltpu.roll(x_ref[pl.ds(a8, win), :], (win - d) % win,
                              axis=0)
            g = xwin[:nb] + jnp.dot(
                attn.astype(jnp.bfloat16), wa_ref[...],
                preferred_element_type=jnp.float32)
            zr = jax.nn.sigmoid(g[:, : 2 * H] + hm[:, nb * L:])
            z = zr[:, :H]
            r = zr[:, H:]
            n = jnp.tanh(g[:, 2 * H:] + jnp.dot(
                (r * h).astype(jnp.bfloat16), un_ref[...],
                preferred_element_type=jnp.float32))
            hn = (1.0 - z) * n + z * h
            hsel = jnp.where(lane < b, hn, h)
            owin = pltpu.roll(
                jnp.concatenate([hsel, jnp.zeros((8, H), jnp.float32)],
                                axis=0), d, axis=0)
            keep = (roww >= d) & (roww < d + nb)
            out_ref[pl.ds(a8, win), :] = jnp.where(
                keep, owin, out_ref[pl.ds(a8, win), :])
            return off + b, hsel

        def pair(it, carry):
            off, h = carry
            off, h = substep(2 * it, off, h)
            off, h = substep(2 * it + 1, off, h)
            return off, h

        return pair

    h0 = jnp.zeros((B, H), jnp.float32)
    off, h = jax.lax.fori_loop(0, T // 4, make_pair(B), (jnp.int32(0), h0))
    hf_ref[0, B // 2:, :] = h[B // 2:]
    _, h8 = jax.lax.fori_loop(T // 4, T // 2, make_pair(B // 2),
                              (off, h[: B // 2]))
    hf_ref[0, 0: B // 2, :] = h8


def kernel(pack_data, batch_sizes, context, context_mask, Wq, Wk, Wz, Wr, Wn,
           Uz, Ur, Un, bz, br, bn):
    f32 = jnp.float32
    pack_pad = jnp.zeros((PAD, D), f32).at[:TOTAL].set(pack_data)
    wcat = jnp.concatenate([Wz[:D], Wr[:D], Wn[:D]], axis=1)
    bcat = jnp.concatenate([bz, br, bn])[None, :]
    wa = jnp.concatenate([Wz[D:], Wr[D:], Wn[D:]], axis=1)
    ucat = jnp.concatenate([Uz, Ur], axis=1)
    ctx2 = context.reshape(B * L, CD)
    madd1 = jnp.where(context_mask, 0.0, -1e9).astype(f32)
    madd = jnp.where(jnp.eye(B, dtype=bool)[:, :, None],
                     madd1[:, None, :], -1e9).reshape(B, B * L).astype(f32)

    QKT = pl.pallas_call(
        _qkt_kernel,
        out_shape=jax.ShapeDtypeStruct((H, B * L), jnp.bfloat16),
    )(Wq, Wk, ctx2)
    ucat_b = ucat.astype(jnp.bfloat16)
    hm16 = jnp.concatenate([QKT, ucat_b], axis=1)
    hm8 = jnp.concatenate([QKT[:, : B * L // 2], ucat_b], axis=1)

    X = pl.pallas_call(
        _proj_kernel,
        out_shape=jax.ShapeDtypeStruct((PAD, 3 * H), f32),
        compiler_params=pltpu.CompilerParams(vmem_limit_bytes=100 * 2**20),
    )(pack_pad, wcat, bcat)

    out_pad, hidden_final = pl.pallas_call(
        _loop_kernel,
        out_shape=(jax.ShapeDtypeStruct((PAD, H), f32),
                   jax.ShapeDtypeStruct((1, B, H), f32)),
        compiler_params=pltpu.CompilerParams(vmem_limit_bytes=110 * 2**20),
    )(X, hm16, hm8, ctx2.astype(jnp.bfloat16), madd,
      Un.astype(jnp.bfloat16), wa.astype(jnp.bfloat16))

    return out_pad[:TOTAL], hidden_final


# speedup vs baseline: 10.0169x; 1.0095x over previous
import numpy as np
import jax
import jax.numpy as jnp
from jax.experimental import pallas as pl
from jax.experimental.pallas import tpu as pltpu

D = 512
H = 512
CD = 512
L = 128
B = 16
T = 512
TOTAL = 4352
PAD = TOTAL + B
SCALE = 1.0 / np.sqrt(H)


def _qkt_kernel(wq_ref, wk_ref, c2_ref, o_ref):
    kt = jax.lax.dot_general(
        wk_ref[...], c2_ref[...], (((0,), (1,)), ((), ())),
        preferred_element_type=jnp.float32)
    o_ref[...] = (SCALE * jnp.dot(
        wq_ref[...], kt, preferred_element_type=jnp.float32)
                  ).astype(jnp.bfloat16)


def _proj_kernel(a_ref, b_ref, bias_ref, o_ref):
    o_ref[...] = jnp.dot(a_ref[...], b_ref[...],
                         preferred_element_type=jnp.float32) + bias_ref[...]


def _loop_kernel(x_ref, hm16_ref, hm8_ref, ctx2_ref, madd_ref, un_ref,
                 wa_ref, out_ref, hf_ref):

    def make_pair(nb):
        win = nb + 8
        hm_ref = hm16_ref if nb == B else hm8_ref
        lane = jax.lax.broadcasted_iota(jnp.int32, (nb, 1), 0)
        roww = jax.lax.broadcasted_iota(jnp.int32, (win, 1), 0)

        def substep(t, off, h):
            b = B - t // 32
            a8 = off // 8 * 8
            d = off - a8
            hm = jnp.dot(h.astype(jnp.bfloat16), hm_ref[...],
                         preferred_element_type=jnp.float32)
            s = hm[:, 0:nb * L] + madd_ref[0:nb, 0:nb * L]
            m = jnp.max(s, axis=-1, keepdims=True)
            e = jnp.exp(s - m)
            w = (e / jnp.sum(e, axis=-1, keepdims=True)).astype(jnp.bfloat16)
            attn = jnp.dot(w, ctx2_ref[0:nb * L, :],
                           preferred_element_type=jnp.float32)
            xwin = pltpu.roll(x_ref[pl.ds(a8, win), :], (win - d) % win,
                              axis=0)
            g = xwin[:nb] + jnp.dot(
                attn.astype(jnp.bfloat16), wa_ref[...],
                preferred_element_type=jnp.float32)
            zr = jax.nn.sigmoid(g[:, : 2 * H] + hm[:, nb * L:])
            z = zr[:, :H]
            r = zr[:, H:]
            n = jnp.tanh(g[:, 2 * H:] + jnp.dot(
                (r * h).astype(jnp.bfloat16), un_ref[...],
                preferred_element_type=jnp.float32))
            hn = (1.0 - z) * n + z * h
            hsel = jnp.where(lane < b, hn, h)
            owin = pltpu.roll(
                jnp.concatenate([hsel, jnp.zeros((8, H), jnp.float32)],
                                axis=0), d, axis=0)
            keep = (roww >= d) & (roww < d + nb)
            out_ref[pl.ds(a8, win), :] = jnp.where(
                keep, owin, out_ref[pl.ds(a8, win), :])
            return off + b, hsel

        def quad(it, carry):
            off, h = carry
            off, h = substep(4 * it, off, h)
            off, h = substep(4 * it + 1, off, h)
            off, h = substep(4 * it + 2, off, h)
            off, h = substep(4 * it + 3, off, h)
            return off, h

        return quad

    h0 = jnp.zeros((B, H), jnp.float32)
    off, h = jax.lax.fori_loop(0, T // 8, make_pair(B), (jnp.int32(0), h0))
    hf_ref[0, B // 2:, :] = h[B // 2:]
    _, h8 = jax.lax.fori_loop(T // 8, T // 4, make_pair(B // 2),
                              (off, h[: B // 2]))
    hf_ref[0, 0: B // 2, :] = h8


def kernel(pack_data, batch_sizes, context, context_mask, Wq, Wk, Wz, Wr, Wn,
           Uz, Ur, Un, bz, br, bn):
    f32 = jnp.float32
    pack_pad = jnp.zeros((PAD, D), f32).at[:TOTAL].set(pack_data)
    wcat = jnp.concatenate([Wz[:D], Wr[:D], Wn[:D]], axis=1)
    bcat = jnp.concatenate([bz, br, bn])[None, :]
    wa = jnp.concatenate([Wz[D:], Wr[D:], Wn[D:]], axis=1)
    ucat = jnp.concatenate([Uz, Ur], axis=1)
    ctx2 = context.reshape(B * L, CD)
    madd1 = jnp.where(context_mask, 0.0, -1e9).astype(f32)
    madd = jnp.where(jnp.eye(B, dtype=bool)[:, :, None],
                     madd1[:, None, :], -1e9).reshape(B, B * L).astype(f32)

    QKT = pl.pallas_call(
        _qkt_kernel,
        out_shape=jax.ShapeDtypeStruct((H, B * L), jnp.bfloat16),
    )(Wq, Wk, ctx2)
    ucat_b = ucat.astype(jnp.bfloat16)
    hm16 = jnp.concatenate([QKT, ucat_b], axis=1)
    hm8 = jnp.concatenate([QKT[:, : B * L // 2], ucat_b], axis=1)

    X = pl.pallas_call(
        _proj_kernel,
        out_shape=jax.ShapeDtypeStruct((PAD, 3 * H), f32),
        compiler_params=pltpu.CompilerParams(vmem_limit_bytes=100 * 2**20),
    )(pack_pad, wcat, bcat)

    out_pad, hidden_final = pl.pallas_call(
        _loop_kernel,
        out_shape=(jax.ShapeDtypeStruct((PAD, H), f32),
                   jax.ShapeDtypeStruct((1, B, H), f32)),
        compiler_params=pltpu.CompilerParams(vmem_limit_bytes=110 * 2**20),
    )(X, hm16, hm8, ctx2.astype(jnp.bfloat16), madd,
      Un.astype(jnp.bfloat16), wa.astype(jnp.bfloat16))

    return out_pad[:TOTAL], hidden_final
